# baseline (speedup 1.0000x reference)
import functools

import jax
import jax.numpy as jnp
from jax import lax
from jax.experimental import pallas as pl
from jax.experimental.pallas import tpu as pltpu
from jax.experimental.pallas import tpu_sc as plsc

VOCAB = 256
DIM = 768
NC = 2
NS = 16
NW = NC * NS
NIB = 4
NRB = 2


@functools.lru_cache(maxsize=None)
def _make_emb(batch: int, seq: int):
    IPW = batch // NW
    T0 = seq // 8 * 8
    TCOL = T0 + 8
    NCOL = TCOL + 8
    mesh = plsc.VectorSubcoreMesh(core_axis_name="c", subcore_axis_name="s")

    @functools.partial(
        pl.kernel,
        mesh=mesh,
        out_type=jax.ShapeDtypeStruct((batch, seq, DIM), jnp.float32),
        scratch_types=[
            pltpu.VMEM((NIB, NCOL), jnp.int32),
            pltpu.VMEM((NRB, seq, DIM), jnp.float32),
            pltpu.VMEM((8, DIM), jnp.float32),
        ] + [pltpu.SemaphoreType.DMA] * (NIB + 2 * NRB + 2),
        compiler_params=pltpu.CompilerParams(use_tc_tiling_on_sc=True),
    )
    def emb(ids_hbm, table_hbm, out_hbm, idx_v, rows_v, tail_v, *sems):
        isem = sems[:NIB]
        gsem = sems[NIB:NIB + NRB]
        wasem = sems[NIB + NRB:NIB + 2 * NRB]
        tsem = sems[NIB + 2 * NRB]
        wbsem = sems[NIB + 2 * NRB + 1]
        wid = lax.axis_index("s") * NC + lax.axis_index("c")
        base = wid * IPW
        t0_dyn = pl.multiple_of(lax.axis_index("c") * 0 + T0, 8)

        def idx_load(j, ib):
            return pltpu.make_async_copy(
                ids_hbm.at[base + j], idx_v.at[ib], isem[ib])

        def gather_main(ib, rb):
            return pltpu.make_async_copy(
                table_hbm.at[idx_v.at[ib, pl.ds(0, seq)]],
                rows_v.at[rb], gsem[rb])

        def gather_tail(ib):
            return pltpu.make_async_copy(
                table_hbm.at[idx_v.at[ib, pl.ds(TCOL, 8)]], tail_v, tsem)

        def write_main(j, rb):
            return pltpu.make_async_copy(
                rows_v.at[rb], out_hbm.at[base + j], wasem[rb])

        def write_tail(j):
            return pltpu.make_async_copy(
                tail_v, out_hbm.at[base + j, pl.ds(t0_dyn, 8)], wbsem)

        for k in range(NIB):
            idx_load(k, k).start()
        idx_load(0, 0).wait()
        gather_main(0, 0).start()
        gather_tail(0).start()
        idx_load(0, 1).wait()
        gather_main(1, 1).start()

        def body(i, carry):
            for k in range(NIB):
                j = i * NIB + k
                rb = k % NRB
                gather_main(k, rb).wait()
                write_main(j, rb).start()
                gather_tail(k).wait()
                write_main(j, rb).wait()
                write_tail(j).start()
                write_tail(j).wait()

                @pl.when(j + 1 < IPW)
                def _next_tail():
                    gather_tail((k + 1) % NIB).start()

                @pl.when(j + NIB < IPW)
                def _prefetch_ids():
                    idx_load(j + NIB, k).start()

                @pl.when(j + NRB < IPW)
                def _next_gather():
                    idx_load(0, (k + NRB) % NIB).wait()
                    gather_main((k + NRB) % NIB, rb).start()
            return carry

        lax.fori_loop(0, IPW // NIB, body, 0)

    return emb


def kernel(token_ids, table):
    batch, seq = token_ids.shape
    ids = token_ids.astype(jnp.int32)
    t0 = seq // 8 * 8
    zeros3 = jnp.zeros((batch, t0 + 8 - seq), jnp.int32)
    ids_packed = jnp.concatenate([ids, zeros3, ids[:, t0:], zeros3], axis=1)
    return _make_emb(batch, seq)(ids_packed, table)


# speedup vs baseline: 1.6737x; 1.6737x over previous
import functools

import jax
import jax.numpy as jnp
from jax import lax
from jax.experimental import pallas as pl
from jax.experimental.pallas import tpu as pltpu
from jax.experimental.pallas import tpu_sc as plsc

VOCAB = 256
DIM = 768
NC = 2
NS = 16
NW = NC * NS
NIB = 4
NRB = 2


@functools.lru_cache(maxsize=None)
def _make_emb(batch: int, seq: int):
    IPW = batch // NW
    mesh = plsc.VectorSubcoreMesh(core_axis_name="c", subcore_axis_name="s")

    @functools.partial(
        pl.kernel,
        mesh=mesh,
        out_type=jax.ShapeDtypeStruct((batch, seq, DIM), jnp.float32),
        scratch_types=[
            pltpu.VMEM((NIB, seq), jnp.int32),
            pltpu.VMEM((NRB, seq, DIM), jnp.float32),
        ] + [pltpu.SemaphoreType.DMA] * (NIB + 2 * NRB),
        compiler_params=pltpu.CompilerParams(use_tc_tiling_on_sc=True),
    )
    def emb(ids_hbm, table_hbm, out_hbm, idx_v, rows_v, *sems):
        isem = sems[:NIB]
        gsem = sems[NIB:NIB + NRB]
        wsem = sems[NIB + NRB:]
        wid = lax.axis_index("s") * NC + lax.axis_index("c")
        base = wid * IPW

        def idx_load(j, ib):
            return pltpu.make_async_copy(
                ids_hbm.at[base + j], idx_v.at[ib], isem[ib])

        def gather(ib, rb):
            return pltpu.make_async_copy(
                table_hbm.at[idx_v.at[ib]], rows_v.at[rb], gsem[rb])

        def write(j, rb):
            return pltpu.make_async_copy(
                rows_v.at[rb], out_hbm.at[base + j], wsem[rb])

        for k in range(NIB):
            idx_load(k, k).start()
        for k in range(NRB):
            idx_load(k, k).wait()
            gather(k, k).start()

        def body(i, carry):
            for k in range(NIB):
                j = i * NIB + k
                rb = k % NRB
                gather(k, rb).wait()
                write(j, rb).start()

                @pl.when(j + NIB < IPW)
                def _prefetch_ids():
                    idx_load(j + NIB, k).start()

                @pl.when(j + NRB < IPW)
                def _next_gather():
                    write(j, rb).wait()
                    idx_load(0, (k + NRB) % NIB).wait()
                    gather((k + NRB) % NIB, rb).start()
            return carry

        lax.fori_loop(0, IPW // NIB, body, 0)
        for rb in range(NRB):
            write(0, rb).wait()

    return emb


@functools.lru_cache(maxsize=None)
def _make_tail_fix(batch: int, seq: int):
    t0 = (seq // 8) * 8
    ntail = seq - t0
    BB = 512

    def body(ids_ref, table_ref, big_ref, out_ref):
        del big_ref
        ids = ids_ref[:, t0:seq]
        oh = (ids[..., None] == jax.lax.broadcasted_iota(
            jnp.int32, (1, 1, VOCAB), 2)).astype(jnp.float32)
        rows = jax.lax.dot_general(
            oh, table_ref[...], (((2,), (0,)), ((), ())),
            precision=jax.lax.Precision.HIGHEST)
        out_ref[:, :ntail, :] = rows
        out_ref[:, ntail:, :] = jnp.zeros((BB, 8 - ntail, DIM), jnp.float32)

    return pl.pallas_call(
        body,
        grid=(batch // BB,),
        in_specs=[
            pl.BlockSpec((BB, seq), lambda i: (i, 0)),
            pl.BlockSpec((VOCAB, DIM), lambda i: (0, 0)),
            pl.BlockSpec(memory_space=pltpu.MemorySpace.HBM),
        ],
        out_specs=pl.BlockSpec((BB, 8, DIM), lambda i: (i, seq // 8, 0)),
        out_shape=jax.ShapeDtypeStruct((batch, seq, DIM), jnp.float32),
        input_output_aliases={2: 0},
    )


def kernel(token_ids, table):
    batch, seq = token_ids.shape
    ids = token_ids.astype(jnp.int32)
    out = _make_emb(batch, seq)(ids, table)
    if seq % 8:
        out = _make_tail_fix(batch, seq)(ids, table, out)
    return out
